# baseline (speedup 1.0000x reference)
import jax
import jax.numpy as jnp
from jax.experimental import pallas as pl


_BM = 2000


def _body(x_ref, u_ref, w_ref, o_ref):
    x = x_ref[...]
    u = jnp.clip(u_ref[...], 1e-6, 1.0 - 1e-6)
    s = x - jnp.log(-jnp.log(u))
    d = jax.lax.dot_general(
        s, w_ref[...], (((1,), (0,)), ((), ())),
        preferred_element_type=jnp.float32,
        precision=jax.lax.Precision.HIGHEST,
    )
    o_ref[...] = (d > 0).astype(jnp.int32)


def _pair_diff_weights():
    k = jnp.arange(128)
    w = jnp.zeros((256, 128), jnp.float32)
    w = w.at[2 * k, k].set(-1.0)
    w = w.at[2 * k + 1, k].set(1.0)
    return w


def kernel(edge_logp, noise_u):
    n = edge_logp.shape[0]
    m = n // 128
    x = edge_logp.reshape(m, 256)
    u = noise_u.reshape(m, 256)
    w = _pair_diff_weights()
    grid = m // _BM
    out = pl.pallas_call(
        _body,
        grid=(grid,),
        in_specs=[
            pl.BlockSpec((_BM, 256), lambda i: (i, 0)),
            pl.BlockSpec((_BM, 256), lambda i: (i, 0)),
            pl.BlockSpec((256, 128), lambda i: (0, 0)),
        ],
        out_specs=pl.BlockSpec((_BM, 128), lambda i: (i, 0)),
        out_shape=jax.ShapeDtypeStruct((m, 128), jnp.int32),
    )(x, u, w)
    return out.reshape(n)


# speedup vs baseline: 197.1609x; 197.1609x over previous
import jax
import jax.numpy as jnp
from jax.experimental import pallas as pl


_BR = 1000


def _body(x_ref, u_ref, o_ref):
    x = x_ref[...]
    u = jnp.clip(u_ref[...], 1e-6, 1.0 - 1e-6)
    s = x - jnp.log(-jnp.log(u))
    s3 = s.reshape(s.shape[0] // 2, 2, 128)
    o_ref[...] = (s3[:, 1, :] > s3[:, 0, :]).astype(jnp.int32)


def _flat_view(a, n):
    g = n // 128
    return a.reshape(g, 128, 2).transpose(0, 2, 1).reshape(2 * g, 128)


def kernel(edge_logp, noise_u):
    n = edge_logp.shape[0]
    g = n // 128
    x = _flat_view(edge_logp, n)
    u = _flat_view(noise_u, n)
    grid = g // _BR
    out = pl.pallas_call(
        _body,
        grid=(grid,),
        in_specs=[
            pl.BlockSpec((2 * _BR, 128), lambda i: (i, 0)),
            pl.BlockSpec((2 * _BR, 128), lambda i: (i, 0)),
        ],
        out_specs=pl.BlockSpec((_BR, 128), lambda i: (i, 0)),
        out_shape=jax.ShapeDtypeStruct((g, 128), jnp.int32),
    )(x, u)
    return out.reshape(n)
